# Initial kernel scaffold; baseline (speedup 1.0000x reference)
import functools

import jax
import jax.numpy as jnp
from jax import lax
from jax.experimental import pallas as pl
from jax.experimental.pallas import tpu as pltpu
from jax.experimental.pallas import tpu_sc as plsc

N = 10000
E = 160000
D = 256
NCLS = 4
G = 64

HALF = 128
CHUNK = 128
NSUB = 16
NCORE = 2
NCHUNK = 1280
EPAD = NCHUNK * CHUNK - E
NPAD = 10016
ZPAD = NPAD // NSUB
ZONE = N // NSUB
ROWBLK = 1000

_mesh = plsc.VectorSubcoreMesh(core_axis_name="c", subcore_axis_name="s")


def _sc_degree(dst3, ones16, zeros16):
    @functools.partial(
        pl.kernel,
        out_type=jax.ShapeDtypeStruct((NCORE, N, 16), jnp.float32),
        mesh=_mesh,
        scratch_types=[
            pltpu.VMEM((1, CHUNK), jnp.int32),
            pltpu.VMEM((CHUNK, 16), jnp.float32),
            pltpu.VMEM_SHARED((NPAD, 16), jnp.float32),
        ],
    )
    def deg_kernel(dst_hbm, ones_hbm, zeros_hbm, out_hbm, idx_v, ones_v, shared):
        c = lax.axis_index("c")
        s = lax.axis_index("s")
        pltpu.sync_copy(zeros_hbm, shared.at[pl.ds(s * ZPAD, ZPAD)])
        pltpu.sync_copy(ones_hbm, ones_v)
        plsc.subcore_barrier()
        base = c * (NCHUNK // 2) + s * (NCHUNK // 2 // NSUB)

        @pl.loop(0, NCHUNK // 2 // NSUB)
        def _(i):
            pltpu.sync_copy(dst_hbm.at[base + i], idx_v)
            pltpu.sync_copy(ones_v, shared.at[idx_v.at[0]], add=True)

        plsc.subcore_barrier()
        pltpu.sync_copy(shared.at[pl.ds(s * ZONE, ZONE)],
                        out_hbm.at[c, pl.ds(s * ZONE, ZONE)])

    return deg_kernel(dst3, ones16, zeros16)


def _sc_scatter(g_split, src3, dst3, zeros128):
    @functools.partial(
        pl.kernel,
        out_type=jax.ShapeDtypeStruct((NCORE, N, HALF), jnp.float32),
        mesh=_mesh,
        scratch_types=[
            pltpu.VMEM((1, CHUNK), jnp.int32),
            pltpu.VMEM((1, CHUNK), jnp.int32),
            pltpu.VMEM((CHUNK, HALF), jnp.float32),
            pltpu.VMEM_SHARED((NPAD, HALF), jnp.float32),
            pltpu.SemaphoreType.DMA,
        ],
    )
    def scat_kernel(g_hbm, src_hbm, dst_hbm, zeros_hbm, out_hbm,
                    src_v, dst_v, rows_v, shared, sem):
        c = lax.axis_index("c")
        s = lax.axis_index("s")
        pltpu.sync_copy(zeros_hbm, shared.at[pl.ds(s * ZPAD, ZPAD)])
        plsc.subcore_barrier()
        base = s * (NCHUNK // NSUB)

        @pl.loop(0, NCHUNK // NSUB)
        def _(i):
            pltpu.sync_copy(src_hbm.at[base + i], src_v)
            pltpu.sync_copy(dst_hbm.at[base + i], dst_v)
            pltpu.async_copy(g_hbm.at[c].at[src_v.at[0]], rows_v, sem).wait()
            pltpu.sync_copy(rows_v, shared.at[dst_v.at[0]], add=True)

        plsc.subcore_barrier()
        pltpu.sync_copy(shared.at[pl.ds(s * ZONE, ZONE)],
                        out_hbm.at[c, pl.ds(s * ZONE, ZONE)])

    return scat_kernel(g_split, src3, dst3, zeros128)


def _tc_fuse(Wm, Wm2, W2, W3, W4, bm8, bm28):
    def body(Wm_r, Wm2_r, W2_r, W3_r, W4_r, bm_r, bm2_r,
             M2_r, M3_r, M4_r, c2_r, c3_r, c4_r):
        f32 = jnp.float32
        Wm_ = Wm_r[...]
        M2_r[...] = jnp.dot(Wm_, W2_r[...], preferred_element_type=f32)
        T = jnp.dot(Wm_, Wm2_r[...], preferred_element_type=f32)
        M3_r[...] = jnp.dot(T, W3_r[...], preferred_element_type=f32)
        M4_r[...] = jnp.dot(Wm_, W4_r[...], preferred_element_type=f32)
        c2_r[...] = jnp.dot(bm_r[...], W2_r[...], preferred_element_type=f32)
        t2 = jnp.dot(bm_r[...], Wm2_r[...], preferred_element_type=f32) + bm2_r[...]
        c3_r[...] = jnp.dot(t2, W3_r[...], preferred_element_type=f32)
        c4_r[...] = jnp.dot(bm_r[...], W4_r[...], preferred_element_type=f32)

    shp = jax.ShapeDtypeStruct
    return pl.pallas_call(
        body,
        out_shape=[shp((D, D), jnp.float32)] * 3 + [shp((8, D), jnp.float32)] * 3,
    )(Wm, Wm2, W2, W3, W4, bm8, bm28)


def _tc_round1(x, W1, deg2):
    def body(x_r, W_r, deg_r, g_o, P_o, dis_o):
        deg = deg_r[0, :, 0:1] + deg_r[1, :, 0:1] + 1.0
        dis = lax.rsqrt(deg)
        P = jnp.dot(x_r[...], W_r[...], preferred_element_type=jnp.float32)
        g = P * dis
        g_o[0] = g[:, :HALF]
        g_o[1] = g[:, HALF:]
        P_o[...] = P
        dis_o[...] = dis

    shp = jax.ShapeDtypeStruct
    return pl.pallas_call(
        body,
        grid=(N // ROWBLK,),
        in_specs=[
            pl.BlockSpec((ROWBLK, D), lambda i: (i, 0)),
            pl.BlockSpec((D, D), lambda i: (0, 0)),
            pl.BlockSpec((2, ROWBLK, 16), lambda i: (0, i, 0)),
        ],
        out_specs=[
            pl.BlockSpec((2, ROWBLK, HALF), lambda i: (0, i, 0)),
            pl.BlockSpec((ROWBLK, D), lambda i: (i, 0)),
            pl.BlockSpec((ROWBLK, 1), lambda i: (i, 0)),
        ],
        out_shape=[
            shp((2, N, HALF), jnp.float32),
            shp((N, D), jnp.float32),
            shp((N, 1), jnp.float32),
        ],
    )(x, W1, deg2)


def _tc_round(S, P_prev, dis, b_prev, M, c8):
    def body(S_r, P_r, dis_r, b_r, M_r, c_r, g_o, P_o):
        dis = dis_r[...]
        Sfull = jnp.concatenate([S_r[0], S_r[1]], axis=1)
        h = jnp.maximum(dis * Sfull + (dis * dis) * P_r[...] + b_r[...], 0.0)
        P = jnp.dot(h, M_r[...], preferred_element_type=jnp.float32) + c_r[0:1, :]
        g = P * dis
        g_o[0] = g[:, :HALF]
        g_o[1] = g[:, HALF:]
        P_o[...] = P

    shp = jax.ShapeDtypeStruct
    return pl.pallas_call(
        body,
        grid=(N // ROWBLK,),
        in_specs=[
            pl.BlockSpec((2, ROWBLK, HALF), lambda i: (0, i, 0)),
            pl.BlockSpec((ROWBLK, D), lambda i: (i, 0)),
            pl.BlockSpec((ROWBLK, 1), lambda i: (i, 0)),
            pl.BlockSpec((1, D), lambda i: (0, 0)),
            pl.BlockSpec((D, D), lambda i: (0, 0)),
            pl.BlockSpec((8, D), lambda i: (0, 0)),
        ],
        out_specs=[
            pl.BlockSpec((2, ROWBLK, HALF), lambda i: (0, i, 0)),
            pl.BlockSpec((ROWBLK, D), lambda i: (i, 0)),
        ],
        out_shape=[
            shp((2, N, HALF), jnp.float32),
            shp((N, D), jnp.float32),
        ],
    )(S, P_prev, dis, b_prev, M, c8)


def _tc_tail(S, P_prev, dis, b_prev, Wm2, bm2, batch2, Wl, bl):
    grid = N // ROWBLK

    def body(S_r, P_r, dis_r, b_r, Wm2_r, bm2_r, bat_r, Wl_r, bl_r,
             out_r, sums, counts):
        i = pl.program_id(0)
        dis = dis_r[...]
        Sfull = jnp.concatenate([S_r[0], S_r[1]], axis=1)
        h5 = dis * Sfull + (dis * dis) * P_r[...] + b_r[...]
        h6 = jnp.maximum(
            jnp.dot(h5, Wm2_r[...], preferred_element_type=jnp.float32) + bm2_r[...],
            0.0)
        seg = bat_r[...]
        mask = (seg == lax.broadcasted_iota(jnp.int32, (ROWBLK, G), 1))
        mask = mask.astype(jnp.float32)
        psums = lax.dot_general(mask, h6, (((0,), (0,)), ((), ())),
                                preferred_element_type=jnp.float32)
        pcnt = lax.dot_general(mask, jnp.ones((ROWBLK, 1), jnp.float32),
                               (((0,), (0,)), ((), ())),
                               preferred_element_type=jnp.float32)

        @pl.when(i == 0)
        def _():
            sums[...] = psums
            counts[...] = pcnt

        @pl.when(i > 0)
        def _():
            sums[...] += psums
            counts[...] += pcnt

        pooled = sums[...] / jnp.maximum(counts[...], 1.0)
        out_r[...] = jnp.dot(pooled, Wl_r[...],
                             preferred_element_type=jnp.float32) + bl_r[...]

    return pl.pallas_call(
        body,
        grid=(grid,),
        in_specs=[
            pl.BlockSpec((2, ROWBLK, HALF), lambda i: (0, i, 0)),
            pl.BlockSpec((ROWBLK, D), lambda i: (i, 0)),
            pl.BlockSpec((ROWBLK, 1), lambda i: (i, 0)),
            pl.BlockSpec((1, D), lambda i: (0, 0)),
            pl.BlockSpec((D, D), lambda i: (0, 0)),
            pl.BlockSpec((1, D), lambda i: (0, 0)),
            pl.BlockSpec((ROWBLK, 1), lambda i: (i, 0)),
            pl.BlockSpec((D, NCLS), lambda i: (0, 0)),
            pl.BlockSpec((1, NCLS), lambda i: (0, 0)),
        ],
        out_specs=pl.BlockSpec((G, NCLS), lambda i: (0, 0)),
        out_shape=jax.ShapeDtypeStruct((G, NCLS), jnp.float32),
        scratch_shapes=[
            pltpu.VMEM((G, D), jnp.float32),
            pltpu.VMEM((G, 1), jnp.float32),
        ],
    )(S, P_prev, dis, b_prev, Wm2, bm2, batch2, Wl, bl)


def kernel(x, edge_index, batch, W1, b1, W2, b2, W3, b3, W4, b4,
           Wm, bm, Wm2, bm2, Wl, bl):
    f32 = jnp.float32
    src = edge_index[0].astype(jnp.int32)
    dst = edge_index[1].astype(jnp.int32)
    pad_dst = (N + (jnp.arange(EPAD, dtype=jnp.int32) % (NPAD - N)))
    src3 = jnp.concatenate([src, jnp.zeros((EPAD,), jnp.int32)]) \
              .reshape(NCHUNK, 1, CHUNK)
    dst3 = jnp.concatenate([dst, pad_dst]).reshape(NCHUNK, 1, CHUNK)

    ones16 = jnp.ones((CHUNK, 16), f32)
    zeros16 = jnp.zeros((ZPAD, 16), f32)
    zeros128 = jnp.zeros((ZPAD, HALF), f32)
    bm8 = jnp.broadcast_to(bm.reshape(1, D), (8, D))
    bm28 = jnp.broadcast_to(bm2.reshape(1, D), (8, D))

    M2, M3, M4, c2, c3, c4 = _tc_fuse(Wm, Wm2, W2, W3, W4, bm8, bm28)

    deg2 = _sc_degree(dst3, ones16, zeros16)
    g, P, dis = _tc_round1(x, W1, deg2)
    S = _sc_scatter(g, src3, dst3, zeros128)
    g, P = _tc_round(S, P, dis, b1.reshape(1, D), M2, c2)
    S = _sc_scatter(g, src3, dst3, zeros128)
    g, P = _tc_round(S, P, dis, b2.reshape(1, D), M3, c3)
    S = _sc_scatter(g, src3, dst3, zeros128)
    g, P = _tc_round(S, P, dis, b3.reshape(1, D), M4, c4)
    S = _sc_scatter(g, src3, dst3, zeros128)
    g, P = _tc_round(S, P, dis, b4.reshape(1, D), M4, c4)
    S = _sc_scatter(g, src3, dst3, zeros128)

    out = _tc_tail(S, P, dis, b4.reshape(1, D), Wm2, bm2.reshape(1, D),
                   batch.reshape(N, 1).astype(jnp.int32), Wl,
                   bl.reshape(1, NCLS))
    return out


# speedup vs baseline: 5.1882x; 5.1882x over previous
import functools

import jax
import jax.numpy as jnp
from jax import lax
from jax.experimental import pallas as pl
from jax.experimental.pallas import tpu as pltpu
from jax.experimental.pallas import tpu_sc as plsc

N = 10000
E = 160000
D = 256
NCLS = 4
G = 64

HALF = 128
CHUNK = 128
NSUB = 16
NCORE = 2
NCHUNK = 1280
EPAD = NCHUNK * CHUNK - E
NPAD = 10240
ZPAD = NPAD // NSUB
ROWBLK = 1000

_mesh = plsc.VectorSubcoreMesh(core_axis_name="c", subcore_axis_name="s")


def _sc_degree(dst3, ones16, zeros16):
    @functools.partial(
        pl.kernel,
        out_type=jax.ShapeDtypeStruct((NCORE, NPAD, 16), jnp.float32),
        mesh=_mesh,
        scratch_types=[
            pltpu.VMEM((1, CHUNK), jnp.int32),
            pltpu.VMEM((CHUNK, 16), jnp.float32),
            pltpu.VMEM_SHARED((NPAD, 16), jnp.float32),
        ],
    )
    def deg_kernel(dst_hbm, ones_hbm, zeros_hbm, out_hbm, idx_v, ones_v, shared):
        c = lax.axis_index("c")
        s = lax.axis_index("s")
        pltpu.sync_copy(zeros_hbm, shared.at[pl.ds(s * ZPAD, ZPAD)])
        pltpu.sync_copy(ones_hbm, ones_v)
        plsc.subcore_barrier()
        base = c * (NCHUNK // 2) + s * (NCHUNK // 2 // NSUB)

        @pl.loop(0, NCHUNK // 2 // NSUB)
        def _(i):
            pltpu.sync_copy(dst_hbm.at[base + i], idx_v)
            pltpu.sync_copy(ones_v, shared.at[idx_v.at[0]], add=True)

        plsc.subcore_barrier()
        pltpu.sync_copy(shared.at[pl.ds(s * ZPAD, ZPAD)],
                        out_hbm.at[c, pl.ds(s * ZPAD, ZPAD)])

    return deg_kernel(dst3, ones16, zeros16)


def _sc_scatter(g_split, src3, dst3, zeros128):
    @functools.partial(
        pl.kernel,
        out_type=jax.ShapeDtypeStruct((NCORE, NPAD, HALF), jnp.float32),
        mesh=_mesh,
        scratch_types=[
            pltpu.VMEM((1, CHUNK), jnp.int32),
            pltpu.VMEM((1, CHUNK), jnp.int32),
            pltpu.VMEM((CHUNK, HALF), jnp.float32),
            pltpu.VMEM_SHARED((NPAD, HALF), jnp.float32),
            pltpu.SemaphoreType.DMA,
        ],
    )
    def scat_kernel(g_hbm, src_hbm, dst_hbm, zeros_hbm, out_hbm,
                    src_v, dst_v, rows_v, shared, sem):
        c = lax.axis_index("c")
        s = lax.axis_index("s")
        pltpu.sync_copy(zeros_hbm, shared.at[pl.ds(s * ZPAD, ZPAD)])
        plsc.subcore_barrier()
        base = s * (NCHUNK // NSUB)

        @pl.loop(0, NCHUNK // NSUB)
        def _(i):
            pltpu.sync_copy(src_hbm.at[base + i], src_v)
            pltpu.sync_copy(dst_hbm.at[base + i], dst_v)
            pltpu.async_copy(g_hbm.at[c].at[src_v.at[0]], rows_v, sem).wait()
            pltpu.sync_copy(rows_v, shared.at[dst_v.at[0]], add=True)

        plsc.subcore_barrier()
        pltpu.sync_copy(shared.at[pl.ds(s * ZPAD, ZPAD)],
                        out_hbm.at[c, pl.ds(s * ZPAD, ZPAD)])

    return scat_kernel(g_split, src3, dst3, zeros128)


def _tc_fuse(Wm, Wm2, W2, W3, W4, bm8, bm28):
    def body(Wm_r, Wm2_r, W2_r, W3_r, W4_r, bm_r, bm2_r,
             M2_r, M3_r, M4_r, c2_r, c3_r, c4_r):
        f32 = jnp.float32
        Wm_ = Wm_r[...]
        M2_r[...] = jnp.dot(Wm_, W2_r[...], preferred_element_type=f32)
        T = jnp.dot(Wm_, Wm2_r[...], preferred_element_type=f32)
        M3_r[...] = jnp.dot(T, W3_r[...], preferred_element_type=f32)
        M4_r[...] = jnp.dot(Wm_, W4_r[...], preferred_element_type=f32)
        c2_r[...] = jnp.dot(bm_r[...], W2_r[...], preferred_element_type=f32)
        t2 = jnp.dot(bm_r[...], Wm2_r[...], preferred_element_type=f32) + bm2_r[...]
        c3_r[...] = jnp.dot(t2, W3_r[...], preferred_element_type=f32)
        c4_r[...] = jnp.dot(bm_r[...], W4_r[...], preferred_element_type=f32)

    shp = jax.ShapeDtypeStruct
    return pl.pallas_call(
        body,
        out_shape=[shp((D, D), jnp.float32)] * 3 + [shp((8, D), jnp.float32)] * 3,
    )(Wm, Wm2, W2, W3, W4, bm8, bm28)


def _tc_round1(x, W1, deg2):
    def body(x_r, W_r, deg_r, g_o, P_o, dis_o):
        deg = deg_r[0, :, 0:1] + deg_r[1, :, 0:1] + 1.0
        dis = lax.rsqrt(deg)
        P = jnp.dot(x_r[...], W_r[...], preferred_element_type=jnp.float32)
        g = P * dis
        g_o[0] = g[:, :HALF]
        g_o[1] = g[:, HALF:]
        P_o[...] = P
        dis_o[...] = dis

    shp = jax.ShapeDtypeStruct
    return pl.pallas_call(
        body,
        grid=(N // ROWBLK,),
        in_specs=[
            pl.BlockSpec((ROWBLK, D), lambda i: (i, 0)),
            pl.BlockSpec((D, D), lambda i: (0, 0)),
            pl.BlockSpec((2, ROWBLK, 16), lambda i: (0, i, 0)),
        ],
        out_specs=[
            pl.BlockSpec((2, ROWBLK, HALF), lambda i: (0, i, 0)),
            pl.BlockSpec((ROWBLK, D), lambda i: (i, 0)),
            pl.BlockSpec((ROWBLK, 1), lambda i: (i, 0)),
        ],
        out_shape=[
            shp((2, N, HALF), jnp.float32),
            shp((N, D), jnp.float32),
            shp((N, 1), jnp.float32),
        ],
    )(x, W1, deg2)


def _tc_round(S, P_prev, dis, b_prev, M, c8):
    def body(S_r, P_r, dis_r, b_r, M_r, c_r, g_o, P_o):
        dis = dis_r[...]
        Sfull = jnp.concatenate([S_r[0], S_r[1]], axis=1)
        h = jnp.maximum(dis * Sfull + (dis * dis) * P_r[...] + b_r[...], 0.0)
        P = jnp.dot(h, M_r[...], preferred_element_type=jnp.float32) + c_r[0:1, :]
        g = P * dis
        g_o[0] = g[:, :HALF]
        g_o[1] = g[:, HALF:]
        P_o[...] = P

    shp = jax.ShapeDtypeStruct
    return pl.pallas_call(
        body,
        grid=(N // ROWBLK,),
        in_specs=[
            pl.BlockSpec((2, ROWBLK, HALF), lambda i: (0, i, 0)),
            pl.BlockSpec((ROWBLK, D), lambda i: (i, 0)),
            pl.BlockSpec((ROWBLK, 1), lambda i: (i, 0)),
            pl.BlockSpec((1, D), lambda i: (0, 0)),
            pl.BlockSpec((D, D), lambda i: (0, 0)),
            pl.BlockSpec((8, D), lambda i: (0, 0)),
        ],
        out_specs=[
            pl.BlockSpec((2, ROWBLK, HALF), lambda i: (0, i, 0)),
            pl.BlockSpec((ROWBLK, D), lambda i: (i, 0)),
        ],
        out_shape=[
            shp((2, N, HALF), jnp.float32),
            shp((N, D), jnp.float32),
        ],
    )(S, P_prev, dis, b_prev, M, c8)


def _tc_tail(S, P_prev, dis, b_prev, Wm2, bm2, batch2, Wl, bl):
    grid = N // ROWBLK

    def body(S_r, P_r, dis_r, b_r, Wm2_r, bm2_r, bat_r, Wl_r, bl_r,
             out_r, sums, counts):
        i = pl.program_id(0)
        dis = dis_r[...]
        Sfull = jnp.concatenate([S_r[0], S_r[1]], axis=1)
        h5 = dis * Sfull + (dis * dis) * P_r[...] + b_r[...]
        h6 = jnp.maximum(
            jnp.dot(h5, Wm2_r[...], preferred_element_type=jnp.float32) + bm2_r[...],
            0.0)
        seg = bat_r[...]
        mask = (seg == lax.broadcasted_iota(jnp.int32, (ROWBLK, G), 1))
        mask = mask.astype(jnp.float32)
        psums = lax.dot_general(mask, h6, (((0,), (0,)), ((), ())),
                                preferred_element_type=jnp.float32)
        pcnt = lax.dot_general(mask, jnp.ones((ROWBLK, 1), jnp.float32),
                               (((0,), (0,)), ((), ())),
                               preferred_element_type=jnp.float32)

        @pl.when(i == 0)
        def _():
            sums[...] = psums
            counts[...] = pcnt

        @pl.when(i > 0)
        def _():
            sums[...] += psums
            counts[...] += pcnt

        pooled = sums[...] / jnp.maximum(counts[...], 1.0)
        out_r[...] = jnp.dot(pooled, Wl_r[...],
                             preferred_element_type=jnp.float32) + bl_r[...]

    return pl.pallas_call(
        body,
        grid=(grid,),
        in_specs=[
            pl.BlockSpec((2, ROWBLK, HALF), lambda i: (0, i, 0)),
            pl.BlockSpec((ROWBLK, D), lambda i: (i, 0)),
            pl.BlockSpec((ROWBLK, 1), lambda i: (i, 0)),
            pl.BlockSpec((1, D), lambda i: (0, 0)),
            pl.BlockSpec((D, D), lambda i: (0, 0)),
            pl.BlockSpec((1, D), lambda i: (0, 0)),
            pl.BlockSpec((ROWBLK, 1), lambda i: (i, 0)),
            pl.BlockSpec((D, NCLS), lambda i: (0, 0)),
            pl.BlockSpec((1, NCLS), lambda i: (0, 0)),
        ],
        out_specs=pl.BlockSpec((G, NCLS), lambda i: (0, 0)),
        out_shape=jax.ShapeDtypeStruct((G, NCLS), jnp.float32),
        scratch_shapes=[
            pltpu.VMEM((G, D), jnp.float32),
            pltpu.VMEM((G, 1), jnp.float32),
        ],
    )(S, P_prev, dis, b_prev, Wm2, bm2, batch2, Wl, bl)


def kernel(x, edge_index, batch, W1, b1, W2, b2, W3, b3, W4, b4,
           Wm, bm, Wm2, bm2, Wl, bl):
    f32 = jnp.float32
    src = edge_index[0].astype(jnp.int32)
    dst = edge_index[1].astype(jnp.int32)
    pad_dst = (N + (jnp.arange(EPAD, dtype=jnp.int32) % (NPAD - N)))
    src3 = jnp.concatenate([src, jnp.zeros((EPAD,), jnp.int32)]) \
              .reshape(NCHUNK, 1, CHUNK)
    dst3 = jnp.concatenate([dst, pad_dst]).reshape(NCHUNK, 1, CHUNK)

    ones16 = jnp.ones((CHUNK, 16), f32)
    zeros16 = jnp.zeros((ZPAD, 16), f32)
    zeros128 = jnp.zeros((ZPAD, HALF), f32)
    bm8 = jnp.broadcast_to(bm.reshape(1, D), (8, D))
    bm28 = jnp.broadcast_to(bm2.reshape(1, D), (8, D))

    M2, M3, M4, c2, c3, c4 = _tc_fuse(Wm, Wm2, W2, W3, W4, bm8, bm28)

    deg2 = _sc_degree(dst3, ones16, zeros16)
    g, P, dis = _tc_round1(x, W1, deg2)
    S = _sc_scatter(g, src3, dst3, zeros128)
    g, P = _tc_round(S, P, dis, b1.reshape(1, D), M2, c2)
    S = _sc_scatter(g, src3, dst3, zeros128)
    g, P = _tc_round(S, P, dis, b2.reshape(1, D), M3, c3)
    S = _sc_scatter(g, src3, dst3, zeros128)
    g, P = _tc_round(S, P, dis, b3.reshape(1, D), M4, c4)
    S = _sc_scatter(g, src3, dst3, zeros128)
    g, P = _tc_round(S, P, dis, b4.reshape(1, D), M4, c4)
    S = _sc_scatter(g, src3, dst3, zeros128)

    out = _tc_tail(S, P, dis, b4.reshape(1, D), Wm2, bm2.reshape(1, D),
                   batch.reshape(N, 1).astype(jnp.int32), Wl,
                   bl.reshape(1, NCLS))
    return out


# speedup vs baseline: 5.8546x; 1.1284x over previous
import functools

import jax
import jax.numpy as jnp
from jax import lax
from jax.experimental import pallas as pl
from jax.experimental.pallas import tpu as pltpu
from jax.experimental.pallas import tpu_sc as plsc

N = 10000
E = 160000
D = 256
NCLS = 4
G = 64

HALF = 128
CHUNK = 128
NSUB = 16
NCORE = 2
NCHUNK = 1280
EPAD = NCHUNK * CHUNK - E
NPAD = 10240
ZPAD = NPAD // NSUB
ROWBLK = 1000

_mesh = plsc.VectorSubcoreMesh(core_axis_name="c", subcore_axis_name="s")


def _sc_degree(dst3, ones16, zeros16):
    @functools.partial(
        pl.kernel,
        out_type=jax.ShapeDtypeStruct((NCORE, NPAD, 16), jnp.float32),
        mesh=_mesh,
        scratch_types=[
            pltpu.VMEM((1, CHUNK), jnp.int32),
            pltpu.VMEM((CHUNK, 16), jnp.float32),
            pltpu.VMEM_SHARED((NPAD, 16), jnp.float32),
        ],
    )
    def deg_kernel(dst_hbm, ones_hbm, zeros_hbm, out_hbm, idx_v, ones_v, shared):
        c = lax.axis_index("c")
        s = lax.axis_index("s")
        pltpu.sync_copy(zeros_hbm, shared.at[pl.ds(s * ZPAD, ZPAD)])
        pltpu.sync_copy(ones_hbm, ones_v)
        plsc.subcore_barrier()
        base = c * (NCHUNK // 2) + s * (NCHUNK // 2 // NSUB)

        @pl.loop(0, NCHUNK // 2 // NSUB)
        def _(i):
            pltpu.sync_copy(dst_hbm.at[base + i], idx_v)
            pltpu.sync_copy(ones_v, shared.at[idx_v.at[0]], add=True)

        plsc.subcore_barrier()
        pltpu.sync_copy(shared.at[pl.ds(s * ZPAD, ZPAD)],
                        out_hbm.at[c, pl.ds(s * ZPAD, ZPAD)])

    return deg_kernel(dst3, ones16, zeros16)


def _sc_scatter(g_split, idx3, zeros128):
    NCH = NCHUNK // NSUB

    @functools.partial(
        pl.kernel,
        out_type=jax.ShapeDtypeStruct((NCORE, NPAD, HALF), jnp.float32),
        mesh=_mesh,
        scratch_types=[
            pltpu.VMEM((2, CHUNK), jnp.int32),
            pltpu.VMEM((2, CHUNK), jnp.int32),
            pltpu.VMEM((CHUNK, HALF), jnp.float32),
            pltpu.VMEM((CHUNK, HALF), jnp.float32),
            pltpu.VMEM_SHARED((NPAD, HALF), jnp.float32),
            pltpu.SemaphoreType.DMA,
            pltpu.SemaphoreType.DMA,
            pltpu.SemaphoreType.DMA,
        ],
    )
    def scat_kernel(g_hbm, idx_hbm, zeros_hbm, out_hbm,
                    idx_a, idx_b, rows_a, rows_b, shared,
                    sem_i, sem_g, sem_s):
        c = lax.axis_index("c")
        s = lax.axis_index("s")
        base = s * NCH
        pltpu.sync_copy(zeros_hbm, shared.at[pl.ds(s * ZPAD, ZPAD)])
        plsc.subcore_barrier()

        table = g_hbm.at[c]

        @pl.loop(0, NCH // 2)
        def _(j):
            i0 = base + 2 * j
            da = pltpu.async_copy(idx_hbm.at[i0], idx_a, sem_i)
            db = pltpu.async_copy(idx_hbm.at[i0 + 1], idx_b, sem_i)
            da.wait()
            db.wait()
            ga = pltpu.async_copy(table.at[idx_a.at[0]], rows_a, sem_g)
            gb = pltpu.async_copy(table.at[idx_b.at[0]], rows_b, sem_g)
            ga.wait()
            gb.wait()
            sa = pltpu.async_copy(rows_a, shared.at[idx_a.at[1]], sem_s, add=True)
            sb = pltpu.async_copy(rows_b, shared.at[idx_b.at[1]], sem_s, add=True)
            sa.wait()
            sb.wait()

        plsc.subcore_barrier()
        pltpu.sync_copy(shared.at[pl.ds(s * ZPAD, ZPAD)],
                        out_hbm.at[c, pl.ds(s * ZPAD, ZPAD)])

    return scat_kernel(g_split, idx3, zeros128)


def _tc_fuse(Wm, Wm2, W2, W3, W4, bm8, bm28):
    def body(Wm_r, Wm2_r, W2_r, W3_r, W4_r, bm_r, bm2_r,
             M2_r, M3_r, M4_r, c2_r, c3_r, c4_r):
        f32 = jnp.float32
        Wm_ = Wm_r[...]
        M2_r[...] = jnp.dot(Wm_, W2_r[...], preferred_element_type=f32)
        T = jnp.dot(Wm_, Wm2_r[...], preferred_element_type=f32)
        M3_r[...] = jnp.dot(T, W3_r[...], preferred_element_type=f32)
        M4_r[...] = jnp.dot(Wm_, W4_r[...], preferred_element_type=f32)
        c2_r[...] = jnp.dot(bm_r[...], W2_r[...], preferred_element_type=f32)
        t2 = jnp.dot(bm_r[...], Wm2_r[...], preferred_element_type=f32) + bm2_r[...]
        c3_r[...] = jnp.dot(t2, W3_r[...], preferred_element_type=f32)
        c4_r[...] = jnp.dot(bm_r[...], W4_r[...], preferred_element_type=f32)

    shp = jax.ShapeDtypeStruct
    return pl.pallas_call(
        body,
        out_shape=[shp((D, D), jnp.float32)] * 3 + [shp((8, D), jnp.float32)] * 3,
    )(Wm, Wm2, W2, W3, W4, bm8, bm28)


def _tc_round1(x, W1, deg2):
    def body(x_r, W_r, deg_r, g_o, P_o, dis_o):
        deg = deg_r[0, :, 0:1] + deg_r[1, :, 0:1] + 1.0
        dis = lax.rsqrt(deg)
        P = jnp.dot(x_r[...], W_r[...], preferred_element_type=jnp.float32)
        g = P * dis
        g_o[0] = g[:, :HALF]
        g_o[1] = g[:, HALF:]
        P_o[...] = P
        dis_o[...] = dis

    shp = jax.ShapeDtypeStruct
    return pl.pallas_call(
        body,
        grid=(N // ROWBLK,),
        in_specs=[
            pl.BlockSpec((ROWBLK, D), lambda i: (i, 0)),
            pl.BlockSpec((D, D), lambda i: (0, 0)),
            pl.BlockSpec((2, ROWBLK, 16), lambda i: (0, i, 0)),
        ],
        out_specs=[
            pl.BlockSpec((2, ROWBLK, HALF), lambda i: (0, i, 0)),
            pl.BlockSpec((ROWBLK, D), lambda i: (i, 0)),
            pl.BlockSpec((ROWBLK, 1), lambda i: (i, 0)),
        ],
        out_shape=[
            shp((2, N, HALF), jnp.float32),
            shp((N, D), jnp.float32),
            shp((N, 1), jnp.float32),
        ],
    )(x, W1, deg2)


def _tc_round(S, P_prev, dis, b_prev, M, c8):
    def body(S_r, P_r, dis_r, b_r, M_r, c_r, g_o, P_o):
        dis = dis_r[...]
        Sfull = jnp.concatenate([S_r[0], S_r[1]], axis=1)
        h = jnp.maximum(dis * Sfull + (dis * dis) * P_r[...] + b_r[...], 0.0)
        P = jnp.dot(h, M_r[...], preferred_element_type=jnp.float32) + c_r[0:1, :]
        g = P * dis
        g_o[0] = g[:, :HALF]
        g_o[1] = g[:, HALF:]
        P_o[...] = P

    shp = jax.ShapeDtypeStruct
    return pl.pallas_call(
        body,
        grid=(N // ROWBLK,),
        in_specs=[
            pl.BlockSpec((2, ROWBLK, HALF), lambda i: (0, i, 0)),
            pl.BlockSpec((ROWBLK, D), lambda i: (i, 0)),
            pl.BlockSpec((ROWBLK, 1), lambda i: (i, 0)),
            pl.BlockSpec((1, D), lambda i: (0, 0)),
            pl.BlockSpec((D, D), lambda i: (0, 0)),
            pl.BlockSpec((8, D), lambda i: (0, 0)),
        ],
        out_specs=[
            pl.BlockSpec((2, ROWBLK, HALF), lambda i: (0, i, 0)),
            pl.BlockSpec((ROWBLK, D), lambda i: (i, 0)),
        ],
        out_shape=[
            shp((2, N, HALF), jnp.float32),
            shp((N, D), jnp.float32),
        ],
    )(S, P_prev, dis, b_prev, M, c8)


def _tc_tail(S, P_prev, dis, b_prev, Wm2, bm2, batch2, Wl, bl):
    grid = N // ROWBLK

    def body(S_r, P_r, dis_r, b_r, Wm2_r, bm2_r, bat_r, Wl_r, bl_r,
             out_r, sums, counts):
        i = pl.program_id(0)
        dis = dis_r[...]
        Sfull = jnp.concatenate([S_r[0], S_r[1]], axis=1)
        h5 = dis * Sfull + (dis * dis) * P_r[...] + b_r[...]
        h6 = jnp.maximum(
            jnp.dot(h5, Wm2_r[...], preferred_element_type=jnp.float32) + bm2_r[...],
            0.0)
        seg = bat_r[...]
        mask = (seg == lax.broadcasted_iota(jnp.int32, (ROWBLK, G), 1))
        mask = mask.astype(jnp.float32)
        psums = lax.dot_general(mask, h6, (((0,), (0,)), ((), ())),
                                preferred_element_type=jnp.float32)
        pcnt = lax.dot_general(mask, jnp.ones((ROWBLK, 1), jnp.float32),
                               (((0,), (0,)), ((), ())),
                               preferred_element_type=jnp.float32)

        @pl.when(i == 0)
        def _():
            sums[...] = psums
            counts[...] = pcnt

        @pl.when(i > 0)
        def _():
            sums[...] += psums
            counts[...] += pcnt

        pooled = sums[...] / jnp.maximum(counts[...], 1.0)
        out_r[...] = jnp.dot(pooled, Wl_r[...],
                             preferred_element_type=jnp.float32) + bl_r[...]

    return pl.pallas_call(
        body,
        grid=(grid,),
        in_specs=[
            pl.BlockSpec((2, ROWBLK, HALF), lambda i: (0, i, 0)),
            pl.BlockSpec((ROWBLK, D), lambda i: (i, 0)),
            pl.BlockSpec((ROWBLK, 1), lambda i: (i, 0)),
            pl.BlockSpec((1, D), lambda i: (0, 0)),
            pl.BlockSpec((D, D), lambda i: (0, 0)),
            pl.BlockSpec((1, D), lambda i: (0, 0)),
            pl.BlockSpec((ROWBLK, 1), lambda i: (i, 0)),
            pl.BlockSpec((D, NCLS), lambda i: (0, 0)),
            pl.BlockSpec((1, NCLS), lambda i: (0, 0)),
        ],
        out_specs=pl.BlockSpec((G, NCLS), lambda i: (0, 0)),
        out_shape=jax.ShapeDtypeStruct((G, NCLS), jnp.float32),
        scratch_shapes=[
            pltpu.VMEM((G, D), jnp.float32),
            pltpu.VMEM((G, 1), jnp.float32),
        ],
    )(S, P_prev, dis, b_prev, Wm2, bm2, batch2, Wl, bl)


def kernel(x, edge_index, batch, W1, b1, W2, b2, W3, b3, W4, b4,
           Wm, bm, Wm2, bm2, Wl, bl):
    f32 = jnp.float32
    src = edge_index[0].astype(jnp.int32)
    dst = edge_index[1].astype(jnp.int32)
    pad_dst = (N + (jnp.arange(EPAD, dtype=jnp.int32) % (NPAD - N)))
    src_p = jnp.concatenate([src, jnp.zeros((EPAD,), jnp.int32)])
    dst_p = jnp.concatenate([dst, pad_dst])
    dst3 = dst_p.reshape(NCHUNK, 1, CHUNK)
    idx3 = jnp.stack([src_p.reshape(NCHUNK, CHUNK),
                      dst_p.reshape(NCHUNK, CHUNK)], axis=1)

    ones16 = jnp.ones((CHUNK, 16), f32)
    zeros16 = jnp.zeros((ZPAD, 16), f32)
    zeros128 = jnp.zeros((ZPAD, HALF), f32)
    bm8 = jnp.broadcast_to(bm.reshape(1, D), (8, D))
    bm28 = jnp.broadcast_to(bm2.reshape(1, D), (8, D))

    M2, M3, M4, c2, c3, c4 = _tc_fuse(Wm, Wm2, W2, W3, W4, bm8, bm28)

    deg2 = _sc_degree(dst3, ones16, zeros16)
    g, P, dis = _tc_round1(x, W1, deg2)
    S = _sc_scatter(g, idx3, zeros128)
    g, P = _tc_round(S, P, dis, b1.reshape(1, D), M2, c2)
    S = _sc_scatter(g, idx3, zeros128)
    g, P = _tc_round(S, P, dis, b2.reshape(1, D), M3, c3)
    S = _sc_scatter(g, idx3, zeros128)
    g, P = _tc_round(S, P, dis, b3.reshape(1, D), M4, c4)
    S = _sc_scatter(g, idx3, zeros128)
    g, P = _tc_round(S, P, dis, b4.reshape(1, D), M4, c4)
    S = _sc_scatter(g, idx3, zeros128)

    out = _tc_tail(S, P, dis, b4.reshape(1, D), Wm2, bm2.reshape(1, D),
                   batch.reshape(N, 1).astype(jnp.int32), Wl,
                   bl.reshape(1, NCLS))
    return out
